# baseline (speedup 1.0000x reference)
import functools

import jax
import jax.numpy as jnp
from jax import lax
from jax.experimental import pallas as pl
from jax.experimental.pallas import tpu as pltpu
from jax.experimental.pallas import tpu_sc as plsc

H = 128
N_NODES = 10000
N_EDGES = 160000
N_GRAPHS = 64

NC = 2
NS = 16
NW = NC * NS
LANES = 16

E_PAD = 163840
E_W = E_PAD // NW
N_PAD = 10240
N_W = N_PAD // NS
CHUNK = 128
N_CHUNKS = E_W // CHUNK

TE = 4096
_GRID_E = E_PAD // TE
EIN_R = 8
E_H = E_PAD // 2
E_WH = E_H // NW
N_CHUNKS_H = E_WH // CHUNK
_GRID_H = E_H // TE


def _sc_gather_body(x0_hbm, x1_hbm, src_hbm, dst_hbm, ea_hbm, batch_hbm, u64_hbm,
                    ein_hbm,
                    x0_v, x1_v, batch_v, u64_v, src_v, dst_v, ea_v,
                    xs0_v, xs1_v, xd0_v, xd1_v, ug_v, ones_v, gsem):
    cid = lax.axis_index("c")
    sid = lax.axis_index("s")
    wid = sid * NC + cid
    base = wid * E_W

    ins = [(x0_hbm, x0_v), (x1_hbm, x1_v), (batch_hbm, batch_v),
           (u64_hbm, u64_v),
           (src_hbm.at[pl.ds(base, E_W)], src_v),
           (dst_hbm.at[pl.ds(base, E_W)], dst_v),
           (ea_hbm.at[pl.ds(base, E_W)], ea_v)]
    for src_r, dst_r in ins:
        pltpu.async_copy(src_r, dst_r, gsem)
    for src_r, dst_r in ins:
        pltpu.make_async_copy(src_r, dst_r, gsem).wait()

    of = jnp.ones((LANES,), jnp.float32)

    def body(i, _):
        for u_ in range(4):
            off = i * (4 * LANES) + u_ * LANES
            s16 = src_v[pl.ds(off, LANES)]
            d16 = dst_v[pl.ds(off, LANES)]
            xs0_v[pl.ds(off, LANES)] = plsc.load_gather(x0_v, [s16])
            xs1_v[pl.ds(off, LANES)] = plsc.load_gather(x1_v, [s16])
            xd0_v[pl.ds(off, LANES)] = plsc.load_gather(x0_v, [d16])
            xd1_v[pl.ds(off, LANES)] = plsc.load_gather(x1_v, [d16])
            b16 = plsc.load_gather(batch_v, [s16])
            ug_v[pl.ds(off, LANES)] = plsc.load_gather(u64_v, [b16])
            ones_v[pl.ds(off, LANES)] = of
        return 0

    lax.fori_loop(0, E_W // (4 * LANES), body, 0)

    outs = [(xs0_v, 0), (xs1_v, 1), (xd0_v, 2), (xd1_v, 3),
            (ea_v, 4), (ug_v, 5), (ones_v, 6), (ones_v, 7)]
    for src_r, row in outs:
        pltpu.async_copy(src_r, ein_hbm.at[row, pl.ds(base, E_W)], gsem)
    for src_r, row in outs:
        pltpu.make_async_copy(src_r, ein_hbm.at[row, pl.ds(base, E_W)],
                              gsem).wait()


@functools.lru_cache(maxsize=None)
def _get_sc_gather():
  return pl.kernel(
    _sc_gather_body,
    out_type=jax.ShapeDtypeStruct((EIN_R, E_PAD), jnp.float32),
    mesh=plsc.VectorSubcoreMesh(core_axis_name="c", subcore_axis_name="s", num_cores=NC, num_subcores=NS),
    scratch_types=[
        pltpu.VMEM((N_NODES,), jnp.float32),
        pltpu.VMEM((N_NODES,), jnp.float32),
        pltpu.VMEM((N_NODES,), jnp.int32),
        pltpu.VMEM((64,), jnp.float32),
        pltpu.VMEM((E_W,), jnp.int32),
        pltpu.VMEM((E_W,), jnp.int32),
        pltpu.VMEM((E_W,), jnp.float32),
        pltpu.VMEM((E_W,), jnp.float32),
        pltpu.VMEM((E_W,), jnp.float32),
        pltpu.VMEM((E_W,), jnp.float32),
        pltpu.VMEM((E_W,), jnp.float32),
        pltpu.VMEM((E_W,), jnp.float32),
        pltpu.VMEM((E_W,), jnp.float32),
        pltpu.SemaphoreType.DMA,
    ],
    compiler_params=pltpu.CompilerParams(needs_layout_passes=False),
    name="sc_gather_edges",
  )


CHUNKB = CHUNK
N_SUPER = E_WH // CHUNKB


def _sc_scatter_body(h_hbm, dst2_hbm, sums_hbm, bufa, bufb, idxa, idxb,
                     sums_sh, sema, semb, semia, semib, ssema, ssemb):
    cid = lax.axis_index("c")
    sid = lax.axis_index("s")
    wid = sid * NC + cid
    base_e = wid * E_WH
    base_c = wid * N_CHUNKS_H
    base_r = sid * N_W

    zf = jnp.zeros((LANES,), jnp.float32)

    def zero_row(r, _):
        for k in range(H // LANES):
            bufa[r, pl.ds(k * LANES, LANES)] = zf
        return 0

    lax.fori_loop(0, CHUNKB, zero_row, 0)

    for k in range(N_W // CHUNKB):
        pltpu.sync_copy(bufa, sums_sh.at[pl.ds(base_r + k * CHUNKB, CHUNKB)])
    plsc.subcore_barrier()

    bufs = (bufa, bufb)
    sems = (sema, semb)
    idxs = (idxa, idxb)
    isems = (semia, semib)
    ssems = (ssema, ssemb)

    def start(s, buf, sem, idx, isem):
        pltpu.async_copy(h_hbm.at[pl.ds(base_e + s * CHUNKB, CHUNKB)], buf, sem)
        pltpu.async_copy(dst2_hbm.at[pl.ds(base_c + s, 1)], idx, isem)

    def wait(s, buf, sem, idx, isem):
        pltpu.make_async_copy(h_hbm.at[pl.ds(base_e + s * CHUNKB, CHUNKB)],
                              buf, sem).wait()
        pltpu.make_async_copy(dst2_hbm.at[pl.ds(base_c + s, 1)],
                              idx, isem).wait()

    start(0, bufa, sema, idxa, semia)

    def super_body(t, _):
        for p in range(2):
            s = t * 2 + p
            buf, sem, idx, isem = bufs[p], sems[p], idxs[p], isems[p]
            ob, os_, oi, ois = bufs[1 - p], sems[1 - p], idxs[1 - p], isems[1 - p]
            ssem, ossem = ssems[p], ssems[1 - p]
            wait(s, buf, sem, idx, isem)

            @pl.when(s >= 1)
            def _():
                pltpu.make_async_copy(ob, sums_sh.at[oi.at[0]], ossem).wait()

            @pl.when(s + 1 < N_SUPER)
            def _():
                start(s + 1, ob, os_, oi, ois)

            pltpu.async_copy(buf, sums_sh.at[idx.at[0]], ssem, add=True)
        return 0

    lax.fori_loop(0, N_SUPER // 2, super_body, 0)
    pltpu.make_async_copy(bufb, sums_sh.at[idxb.at[0]], ssemb).wait()
    plsc.subcore_barrier()

    for k in range(N_W // CHUNKB):
        r = base_r + k * CHUNKB
        pltpu.sync_copy(sums_sh.at[pl.ds(r, CHUNKB)], sums_hbm.at[cid, pl.ds(r, CHUNKB)])


@functools.lru_cache(maxsize=None)
def _get_sc_scatter():
  return pl.kernel(
    _sc_scatter_body,
    out_type=jax.ShapeDtypeStruct((NC, N_PAD, H), jnp.float32),
    mesh=plsc.VectorSubcoreMesh(core_axis_name="c", subcore_axis_name="s", num_cores=NC, num_subcores=NS),
    scratch_types=[
        pltpu.VMEM((CHUNKB, H), jnp.float32),
        pltpu.VMEM((CHUNKB, H), jnp.float32),
        pltpu.VMEM((1, CHUNK), jnp.int32),
        pltpu.VMEM((1, CHUNK), jnp.int32),
        pltpu.VMEM_SHARED((N_PAD, H), jnp.float32),
        pltpu.SemaphoreType.DMA,
        pltpu.SemaphoreType.DMA,
        pltpu.SemaphoreType.DMA,
        pltpu.SemaphoreType.DMA,
        pltpu.SemaphoreType.DMA,
        pltpu.SemaphoreType.DMA,
    ],
    compiler_params=pltpu.CompilerParams(needs_layout_passes=False),
    name="sc_scatter_h",
  )


_N_HI = N_PAD // H


def _tc_cnt_body(dst_row, dst_col, out):
    i = pl.program_id(0)

    @pl.when(i == 0)
    def _():
        out[...] = jnp.zeros((_N_HI, H), jnp.float32)

    oh_hi = (dst_row[...] // H ==
             lax.broadcasted_iota(jnp.int32, (_N_HI, TE), 0)).astype(jnp.float32)
    oh_lo = (dst_col[...] % H ==
             lax.broadcasted_iota(jnp.int32, (TE, H), 1)).astype(jnp.float32)
    out[...] += jnp.dot(oh_hi, oh_lo, preferred_element_type=jnp.float32)


_tc_cnt = pl.pallas_call(
    _tc_cnt_body,
    grid=(_GRID_E,),
    in_specs=[pl.BlockSpec((1, TE), lambda i: (0, i)),
              pl.BlockSpec((TE, 1), lambda i: (i, 0))],
    out_specs=pl.BlockSpec((_N_HI, H), lambda i: (0, 0)),
    out_shape=jax.ShapeDtypeStruct((_N_HI, H), jnp.float32),
    name="tc_counts",
)


def _tc_edge_body(ein,
                  w1e, w2e, b2e, wco,
                  w1n6, w2n, b2n, w3n, b3n,
                  eo_out, h_out):
    f32 = jnp.float32
    tn = (((0,), (0,)), ((), ()))
    e = ein[...]
    h1 = jnp.maximum(
        lax.dot_general(e, w1e[...], tn, preferred_element_type=f32), 0.0)
    h2 = jnp.maximum(
        jnp.dot(h1, w2e[...], preferred_element_type=f32) + b2e[...], 0.0)
    co = jnp.dot(h2, wco[...], preferred_element_type=f32)
    eo_out[...] = co[:, 0:1]

    n1 = (lax.dot_general(e, w1n6[...], tn, preferred_element_type=f32)
          + co[:, 1:])
    n1 = jnp.maximum(n1, 0.0)
    n2 = jnp.maximum(
        jnp.dot(n1, w2n[...], preferred_element_type=f32) + b2n[...], 0.0)
    h_out[...] = jnp.dot(n2, w3n[...], preferred_element_type=f32) + b3n[...]


def _edge_specs(phase):
    def full(shape):
        return pl.BlockSpec(shape, lambda i: tuple(0 for _ in shape))
    off = phase * _GRID_H
    in_specs = [pl.BlockSpec((EIN_R, TE), lambda i: (0, i + off))] + [
        full((EIN_R, H)), full((H, H)), full((1, H)), full((H, 1 + H)),
        full((EIN_R, H)), full((H, H)), full((1, H)), full((H, H)), full((1, H)),
    ]
    out_specs = [pl.BlockSpec((TE, 1), lambda i: (i, 0)),
                 pl.BlockSpec((TE, H), lambda i: (i, 0))]
    return in_specs, out_specs


def _make_tc_edge(phase):
    in_specs, out_specs = _edge_specs(phase)
    return pl.pallas_call(
        _tc_edge_body,
        grid=(_GRID_H,),
        in_specs=in_specs,
        out_specs=out_specs,
        out_shape=(jax.ShapeDtypeStruct((E_H, 1), jnp.float32),
                   jax.ShapeDtypeStruct((E_H, H), jnp.float32)),
        name="tc_edge_mlp%d" % phase,
    )


_tc_edge_a = _make_tc_edge(0)
_tc_edge_b = _make_tc_edge(1)


def _tc_node_body(sums, sums2, cnt, x, batch_c, batch_r, u, u64,
                  wx, wagg, wu, b1, w2, b2,
                  wgu, wgx, bg1, wg2, bg2, wg3, bg3,
                  xn_out, un_out):
    f32 = jnp.float32
    s = (sums[0] + sums[1] + sums2[0] + sums2[1])[:N_NODES, :]
    c = cnt[...][:N_NODES, :]
    agg = s / jnp.maximum(c, 1.0)

    oh = (batch_c[...] == lax.broadcasted_iota(jnp.int32, (N_NODES, N_GRAPHS), 1)
          ).astype(f32)
    ub = jnp.dot(oh, u64[...], preferred_element_type=f32)

    y1 = (jnp.dot(x[...], wx[...], preferred_element_type=f32)
          + jnp.dot(agg, wagg[...], preferred_element_type=f32)
          + ub * wu[...] + b1[...])
    y1 = jnp.maximum(y1, 0.0)
    xn = jnp.dot(y1, w2[...], preferred_element_type=f32) + b2[...]
    xn_out[...] = xn

    oht = (batch_r[...] == lax.broadcasted_iota(jnp.int32, (N_GRAPHS, N_NODES), 0)
           ).astype(f32)
    cg = jnp.sum(oht, axis=1, keepdims=True)
    xm = jnp.dot(oht, xn, preferred_element_type=f32) / jnp.maximum(cg, 1.0)

    g1 = jnp.maximum(jnp.dot(u[...], wgu[...], preferred_element_type=f32)
                     + jnp.dot(xm, wgx[...], preferred_element_type=f32) + bg1[...], 0.0)
    g2 = jnp.maximum(jnp.dot(g1, wg2[...], preferred_element_type=f32) + bg2[...], 0.0)
    un_out[...] = jnp.dot(g2, wg3[...], preferred_element_type=f32) + bg3[...]


def _make_tc_node(outs):
    return pl.pallas_call(
        _tc_node_body,
        out_shape=(jax.ShapeDtypeStruct((N_NODES, 2), jnp.float32),
                   jax.ShapeDtypeStruct((N_GRAPHS, outs), jnp.float32)),
        name="tc_node_global",
    )


_tc_node6 = _make_tc_node(6)
_tc_node1 = _make_tc_node(1)


def kernel(x, edge_attr, u, edge_index, batch, params):
    src = edge_index[0].astype(jnp.int32)
    dst = edge_index[1].astype(jnp.int32)
    pad_e = E_PAD - N_EDGES
    src_p = jnp.concatenate([src, jnp.zeros((pad_e,), jnp.int32)])
    dst_p = jnp.concatenate([dst, jnp.full((pad_e,), N_PAD - 1, jnp.int32)])
    ea = jnp.concatenate([edge_attr.reshape(-1, 1),
                          jnp.zeros((pad_e, 1), jnp.float32)])
    batch = batch.astype(jnp.int32)
    batch_c = batch.reshape(-1, 1)
    batch_r = batch.reshape(1, -1)

    cnt_col = _tc_cnt(dst_p.reshape(1, -1), dst_p.reshape(-1, 1)).reshape(-1, 1)

    dst2a = dst_p[:E_H].reshape(-1, CHUNK)
    dst2b = dst_p[E_H:].reshape(-1, CHUNK)

    prev_b3e = jnp.zeros((1, 1), jnp.float32)
    for i in range(1, 5):
        p = params["layer_%d" % i]
        u64 = u.reshape(-1)[:64]

        ein = _get_sc_gather()(
            x[:, 0], x[:, 1], src_p, dst_p, ea.reshape(-1), batch, u64)

        pe = p["edge"]
        pn1 = p["node_mlp1"]
        w1n = pn1[0]["W"]
        w1nr2 = w1n[2:3, :]
        b3e = pe[2]["b"].reshape(1, 1)
        zrow = jnp.zeros((1, H), jnp.float32)
        w1e8 = jnp.concatenate(
            [pe[0]["W"],
             pe[0]["b"].reshape(1, H) + prev_b3e * pe[0]["W"][4:5, :], zrow])
        w1n8 = jnp.concatenate(
            [w1n[0:2, :], jnp.zeros((4, H), jnp.float32),
             pn1[0]["b"].reshape(1, H) + b3e * w1nr2, zrow])
        wco = jnp.concatenate([pe[2]["W"], pe[2]["W"] @ w1nr2], axis=1)
        ew = (w1e8, pe[1]["W"], pe[1]["b"].reshape(1, H),
              wco,
              w1n8,
              pn1[1]["W"], pn1[1]["b"].reshape(1, H),
              pn1[2]["W"], pn1[2]["b"].reshape(1, H))
        eo_a, h_a = _tc_edge_a(ein, *ew)
        sums_a = _get_sc_scatter()(h_a, dst2a)
        eo_b, h_b = _tc_edge_b(ein, *ew)
        sums_b = _get_sc_scatter()(h_b, dst2b)

        pn2 = p["node_mlp2"]
        pg = p["global"]
        w1 = pn2[0]["W"]
        node_fn = _tc_node6 if i < 4 else _tc_node1
        x, u = node_fn(
            sums_a, sums_b, cnt_col, x, batch_c, batch_r, u, u64.reshape(-1, 1),
            w1[0:2, :], w1[2:2 + H, :], w1[2 + H:3 + H, :],
            pn2[0]["b"].reshape(1, -1),
            pn2[1]["W"], pn2[1]["b"].reshape(1, -1),
            pg[0]["W"][0:6, :], pg[0]["W"][6:8, :], pg[0]["b"].reshape(1, H),
            pg[1]["W"], pg[1]["b"].reshape(1, H),
            pg[2]["W"], pg[2]["b"].reshape(1, -1),
        )
        ea = jnp.concatenate([eo_a, eo_b])
        prev_b3e = b3e

    return u
